# baseline (speedup 1.0000x reference)
import jax
import jax.numpy as jnp
from jax.experimental import pallas as pl
from jax.experimental.pallas import tpu as pltpu

_BLOCK_B = 8192


def _film_body(task_ref, x_ref, g_ref, b_ref, o_ref):
    t = task_ref[0]
    g = g_ref[pl.ds(t, 1), :]
    b = b_ref[pl.ds(t, 1), :]
    o_ref[...] = x_ref[...] * g + b


def kernel(x, gamma_w, beta_w, task_id):
    batch, dim = x.shape
    num_tasks = gamma_w.shape[0]
    task = jnp.asarray(task_id, dtype=jnp.int32).reshape((1,))
    block_b = min(_BLOCK_B, batch)
    grid = (batch // block_b,)
    return pl.pallas_call(
        _film_body,
        grid_spec=pltpu.PrefetchScalarGridSpec(
            num_scalar_prefetch=1,
            grid=grid,
            in_specs=[
                pl.BlockSpec((block_b, dim), lambda i, t: (i, 0)),
                pl.BlockSpec((num_tasks, dim), lambda i, t: (0, 0)),
                pl.BlockSpec((num_tasks, dim), lambda i, t: (0, 0)),
            ],
            out_specs=pl.BlockSpec((block_b, dim), lambda i, t: (i, 0)),
        ),
        out_shape=jax.ShapeDtypeStruct(x.shape, x.dtype),
        compiler_params=pltpu.CompilerParams(
            dimension_semantics=("parallel",),
        ),
    )(task, x, gamma_w, beta_w)
